# Initial kernel scaffold; baseline (speedup 1.0000x reference)
import jax
import jax.numpy as jnp
from jax.experimental import pallas as pl


def kernel(cat_indices, numeric, tables, numeric_kernel, bias):
    raise NotImplementedError("write your pallas kernel here")


# speedup vs baseline: 1.2896x; 1.2896x over previous
import functools

import jax
import jax.numpy as jnp
from jax import lax
from jax.experimental import pallas as pl
from jax.experimental.pallas import tpu as pltpu
from jax.experimental.pallas import tpu_sc as plsc

NUM_CAT = 26
VOCAB = 100000
NUM_NUM = 13
BATCH = 16384

NUM_WORKERS = 32
BPW = BATCH // NUM_WORKERS
IDX_PER_W = BPW * NUM_CAT
NUM_PER_W = BPW * NUM_NUM
GCHUNK = 128
NGATHER = IDX_PER_W // GCHUNK
LANES = 16
GROUPS = BPW // LANES


def _sc_body(tab, idxh, numh, nkbh, out,
             idx_v, vals_v, num_v, nkb_v, out_v,
             sem_idx, sem_num, sem_g):
    wid = lax.axis_index("c") * 16 + lax.axis_index("s")
    base = wid * BPW

    cp_idx = pltpu.make_async_copy(
        idxh.at[pl.ds(wid * IDX_PER_W, IDX_PER_W)], idx_v, sem_idx)
    cp_idx.start()
    cp_num = pltpu.make_async_copy(
        numh.at[pl.ds(wid * NUM_PER_W, NUM_PER_W)], num_v, sem_num)
    cp_num.start()
    cp_nkb = pltpu.make_async_copy(nkbh, nkb_v, sem_num)
    cp_nkb.start()

    cp_idx.wait()

    def issue(t, carry):
        pltpu.make_async_copy(
            tab.at[idx_v.at[pl.ds(t * GCHUNK, GCHUNK)]],
            vals_v.at[pl.ds(t * GCHUNK, GCHUNK)],
            sem_g).start()
        return carry
    lax.fori_loop(0, NGATHER, issue, 0)

    cp_num.wait()
    cp_nkb.wait()
    pltpu.make_async_copy(tab.at[pl.ds(0, IDX_PER_W)], vals_v, sem_g).wait()

    lane = lax.iota(jnp.int32, LANES)
    lane_cat = lane * NUM_CAT
    lane_num = lane * NUM_NUM
    kvecs = [nkb_v[pl.ds(n * LANES, LANES)] for n in range(NUM_NUM + 1)]

    def grp(j, carry):
        acc = kvecs[NUM_NUM]
        cbase = j * (LANES * NUM_CAT)
        for f in range(NUM_CAT):
            acc = acc + plsc.load_gather(vals_v, [lane_cat + (cbase + f)])
        nbase = j * (LANES * NUM_NUM)
        for n in range(NUM_NUM):
            acc = acc + plsc.load_gather(num_v, [lane_num + (nbase + n)]) * kvecs[n]
        out_v[pl.ds(j * LANES, LANES)] = acc
        return carry
    lax.fori_loop(0, GROUPS, grp, 0)

    pltpu.sync_copy(out_v, out.at[pl.ds(base, BPW)])


@jax.jit
def _sc_call(tab_flat, flat_idx, num_flat, nkb):
    mesh = plsc.VectorSubcoreMesh(core_axis_name="c", subcore_axis_name="s")
    f = functools.partial(
        pl.kernel,
        out_type=jax.ShapeDtypeStruct((BATCH,), jnp.float32),
        mesh=mesh,
        compiler_params=pltpu.CompilerParams(needs_layout_passes=False),
        scratch_types=[
            pltpu.VMEM((IDX_PER_W,), jnp.int32),
            pltpu.VMEM((IDX_PER_W,), jnp.float32),
            pltpu.VMEM((NUM_PER_W,), jnp.float32),
            pltpu.VMEM(((NUM_NUM + 1) * LANES,), jnp.float32),
            pltpu.VMEM((BPW,), jnp.float32),
            pltpu.SemaphoreType.DMA,
            pltpu.SemaphoreType.DMA,
            pltpu.SemaphoreType.DMA,
        ],
    )(_sc_body)
    return f(tab_flat, flat_idx, num_flat, nkb)


def kernel(cat_indices, numeric, tables, numeric_kernel, bias):
    cat32 = cat_indices.astype(jnp.int32)
    offs = (jnp.arange(NUM_CAT, dtype=jnp.int32) * VOCAB)[None, :]
    flat_idx = (cat32 + offs).reshape(-1)
    tab_flat = tables.reshape(-1)
    num_flat = numeric.reshape(-1)
    nkb = jnp.concatenate([
        jnp.repeat(numeric_kernel.reshape(-1), LANES),
        jnp.repeat(bias.reshape(-1), LANES),
    ])
    out = _sc_call(tab_flat, flat_idx, num_flat, nkb)
    return out.reshape(BATCH, 1)
